# Initial kernel scaffold; baseline (speedup 1.0000x reference)
import jax
import jax.numpy as jnp
from jax.experimental import pallas as pl


def kernel(embedding, inputs_id, inputs_value):
    raise NotImplementedError("write your pallas kernel here")


# speedup vs baseline: 1.5276x; 1.5276x over previous
import jax
import jax.numpy as jnp
from jax import lax
from jax.experimental import pallas as pl
from jax.experimental.pallas import tpu as pltpu
from jax.experimental.pallas import tpu_sc as plsc

VOCAB = 1000000
D = 32
BATCH = 16384
FIELDS = 26
B = BATCH * FIELDS

NC = 2
NS = 16
NW = NC * NS
PER_W = B // NW
G = 128
K = 13
SC_ROWS = G * K
NSC = PER_W // SC_ROWS
NCH = PER_W // G


def _sc_body(table_hbm, idx_hbm, val_hbm, out_hbm, idx_v, val_v, rows_v, sem):
    wid = lax.axis_index("s") * NC + lax.axis_index("c")
    base = wid * PER_W
    pltpu.sync_copy(idx_hbm.at[wid], idx_v)
    pltpu.sync_copy(val_hbm.at[wid], val_v)

    def superchunk(s, carry):
        copies = []
        for j in range(K):
            copies.append(pltpu.async_copy(
                table_hbm.at[idx_v.at[s * K + j]],
                rows_v.at[pl.ds(j * G, G)],
                sem))
        for c in copies:
            c.wait()

        def group(g, carry2):
            v16 = val_v[pl.ds(s * SC_ROWS + g * 16, 16)]
            for l in range(16):
                i = g * 16 + l
                v = v16[l]
                r0 = rows_v[i, pl.ds(0, 16)]
                rows_v[i, pl.ds(0, 16)] = r0 * v
                r1 = rows_v[i, pl.ds(16, 16)]
                rows_v[i, pl.ds(16, 16)] = r1 * v
            return carry2
        lax.fori_loop(0, SC_ROWS // 16, group, 0)

        pltpu.sync_copy(rows_v,
                        out_hbm.at[pl.ds(base + s * SC_ROWS, SC_ROWS)])
        return carry
    lax.fori_loop(0, NSC, superchunk, 0)


@jax.jit
def kernel(embedding, inputs_id, inputs_value):
    ids = inputs_id.astype(jnp.int32).reshape(NW, NCH, G)
    vals = inputs_value.reshape(NW, PER_W)
    mesh = plsc.VectorSubcoreMesh(core_axis_name="c", subcore_axis_name="s")
    out = pl.kernel(
        _sc_body,
        mesh=mesh,
        compiler_params=pltpu.CompilerParams(use_tc_tiling_on_sc=False),
        out_type=jax.ShapeDtypeStruct((B, D), jnp.float32),
        scratch_types=[
            pltpu.VMEM((NCH, G), jnp.int32),
            pltpu.VMEM((PER_W,), jnp.float32),
            pltpu.VMEM((SC_ROWS, D), jnp.float32),
            pltpu.SemaphoreType.DMA,
        ],
    )(embedding, ids, vals)
    return out.reshape(BATCH, FIELDS, D)
